# Initial kernel scaffold; baseline (speedup 1.0000x reference)
import jax
import jax.numpy as jnp
from jax.experimental import pallas as pl


def kernel(feat, edge, W):
    raise NotImplementedError("write your pallas kernel here")


# speedup vs baseline: 1.0004x; 1.0004x over previous
import functools

import jax
import jax.numpy as jnp
from jax.experimental import pallas as pl
from jax.experimental.pallas import tpu as pltpu

N_NODES = 10000
D_IN = 256
D_OUT = 256
ROW_BLK = 400


def _dense_body(agg_ref, deg_ref, feat_ref, w1t_ref, w2t_ref, out_ref):
    deg = deg_ref[...]
    inv_deg = jnp.where(deg == 0.0, 1.0, 1.0 / deg)
    feat_agg = agg_ref[...] * inv_deg
    h = jnp.dot(feat_agg, w1t_ref[...], preferred_element_type=jnp.float32)
    h += jnp.dot(feat_ref[...], w2t_ref[...], preferred_element_type=jnp.float32)
    h = jnp.maximum(h, 0.0)
    norm = jnp.sqrt(jnp.sum(h * h, axis=1, keepdims=True))
    norm = jnp.maximum(norm, 1e-12)
    out_ref[...] = h / norm


def _dense_stage(agg, deg, feat, w1t, w2t):
    grid = (N_NODES // ROW_BLK,)
    return pl.pallas_call(
        _dense_body,
        grid=grid,
        in_specs=[
            pl.BlockSpec((ROW_BLK, D_IN), lambda i: (i, 0)),
            pl.BlockSpec((ROW_BLK, 1), lambda i: (i, 0)),
            pl.BlockSpec((ROW_BLK, D_IN), lambda i: (i, 0)),
            pl.BlockSpec((D_IN, D_OUT), lambda i: (0, 0)),
            pl.BlockSpec((D_IN, D_OUT), lambda i: (0, 0)),
        ],
        out_specs=pl.BlockSpec((ROW_BLK, D_OUT), lambda i: (i, 0)),
        out_shape=jax.ShapeDtypeStruct((N_NODES, D_OUT), jnp.float32),
    )(agg, deg, feat, w1t, w2t)


def kernel(feat, edge, W):
    src = edge[0]
    dst = edge[1]
    feat_h = jnp.take(feat, src, axis=0)
    agg = jnp.zeros((N_NODES, D_IN), dtype=jnp.float32).at[dst].add(feat_h)
    deg = jnp.bincount(dst, length=N_NODES).astype(jnp.float32)[:, None]
    w1t = W[:, :D_IN].T
    w2t = W[:, D_IN:].T
    return _dense_stage(agg, deg, feat, w1t, w2t)


# speedup vs baseline: 2.6361x; 2.6350x over previous
import functools

import jax
import jax.numpy as jnp
from jax import lax
from jax.experimental import pallas as pl
from jax.experimental.pallas import tpu as pltpu
from jax.experimental.pallas import tpu_sc as plsc

N_NODES = 10000
N_EDGES = 160000
D_IN = 256
D_OUT = 256

NC = 2
NS = 16
DH = 128
CHUNK = 128
E_PAD = 163840
E_TILE = E_PAD // NS
N_CHUNKS = E_TILE // CHUNK
N_PAD = 10240
ROWS_TILE = N_PAD // NS
WCH = 128
HR = N_PAD // DH

ROW_BLK = 400


def _sc_agg_body(featflat, src_hbm, dst_hbm, out_hbm, deg_hbm,
                 sidx_v, didx_v, rows_v, hist_v, agg_sh, sem):
    c = lax.axis_index("c")
    s = lax.axis_index("s")

    zero16 = jnp.zeros((16,), jnp.float32)

    def _zrow(r, _):
        for j in range(DH // 16):
            rows_v[r, pl.ds(j * 16, 16)] = zero16
        return _
    lax.fori_loop(0, CHUNK, _zrow, 0)

    def _zhist(r, _):
        hist_v[pl.ds(r * 16, 16)] = zero16
        return _
    lax.fori_loop(0, N_PAD // 16, _zhist, 0)

    slab0 = s * ROWS_TILE
    for k in range(ROWS_TILE // WCH):
        pltpu.sync_copy(rows_v.at[pl.ds(0, WCH)],
                        agg_sh.at[pl.ds(slab0 + k * WCH, WCH)])
    plsc.subcore_barrier()

    base = s * E_TILE
    coff = c * N_NODES
    ones16 = jnp.ones((16,), jnp.float32)

    def _chunk(g, carry):
        off = base + g * CHUNK
        pltpu.sync_copy(src_hbm.at[pl.ds(off, CHUNK)], sidx_v)
        pltpu.sync_copy(dst_hbm.at[pl.ds(off, CHUNK)], didx_v)
        for j in range(CHUNK // 16):
            sl = pl.ds(j * 16, 16)
            sidx_v[sl] = sidx_v[sl] + coff
        gcp = pltpu.async_copy(featflat.at[sidx_v], rows_v, sem)
        gcp.wait()
        pltpu.sync_copy(rows_v, agg_sh.at[didx_v], add=True)
        return carry
    lax.fori_loop(0, N_CHUNKS, _chunk, 0)
    plsc.subcore_barrier()

    for k in range(ROWS_TILE // WCH):
        r0 = slab0 + k * WCH
        pltpu.sync_copy(agg_sh.at[pl.ds(r0, WCH)], rows_v.at[pl.ds(0, WCH)])
        pltpu.sync_copy(rows_v.at[pl.ds(0, WCH)], out_hbm.at[c, pl.ds(r0, WCH)])

    @pl.when(c == 0)
    def _():
        pltpu.sync_copy(hist_v, deg_hbm.at[s])


_sc_agg = functools.partial(
    pl.kernel,
    out_type=(jax.ShapeDtypeStruct((NC, N_PAD, DH), jnp.float32),
              jax.ShapeDtypeStruct((NS, N_PAD), jnp.float32)),
    mesh=plsc.VectorSubcoreMesh(core_axis_name="c", subcore_axis_name="s"),
    scratch_types=[
        pltpu.VMEM((CHUNK,), jnp.int32),
        pltpu.VMEM((CHUNK,), jnp.int32),
        pltpu.VMEM((CHUNK, DH), jnp.float32),
        pltpu.VMEM((N_PAD,), jnp.float32),
        pltpu.VMEM_SHARED((N_PAD, DH), jnp.float32),
        pltpu.SemaphoreType.DMA,
    ],
)(_sc_agg_body)


def _dense_body(aggA_ref, aggB_ref, deg_ref, feat_ref, w_ref, out_ref):
    deg = jnp.sum(deg_ref[...], axis=1)[:, None]
    inv_deg = jnp.where(deg == 0.0, 1.0, 1.0 / deg)
    dn = (((1,), (1,)), ((), ()))
    h = lax.dot_general(aggA_ref[0] * inv_deg, w_ref[:, :DH], dn,
                        preferred_element_type=jnp.float32)
    h += lax.dot_general(aggB_ref[0] * inv_deg, w_ref[:, DH:D_IN], dn,
                         preferred_element_type=jnp.float32)
    h += lax.dot_general(feat_ref[...], w_ref[:, D_IN:], dn,
                         preferred_element_type=jnp.float32)
    h = jnp.maximum(h, 0.0)
    norm = jnp.maximum(jnp.sqrt(jnp.sum(h * h, axis=1, keepdims=True)), 1e-12)
    out_ref[...] = h / norm


def _dense_stage(agg2, deg, feat, W):
    grid = (N_NODES // ROW_BLK,)
    return pl.pallas_call(
        _dense_body,
        grid=grid,
        in_specs=[
            pl.BlockSpec((1, ROW_BLK, DH), lambda i: (0, i, 0)),
            pl.BlockSpec((1, ROW_BLK, DH), lambda i: (1, i, 0)),
            pl.BlockSpec((ROW_BLK, NS), lambda i: (i, 0)),
            pl.BlockSpec((ROW_BLK, D_IN), lambda i: (i, 0)),
            pl.BlockSpec((D_OUT, 2 * D_IN), lambda i: (0, 0)),
        ],
        out_specs=pl.BlockSpec((ROW_BLK, D_OUT), lambda i: (i, 0)),
        out_shape=jax.ShapeDtypeStruct((N_NODES, D_OUT), jnp.float32),
    )(agg2, agg2, deg, feat, W)


def kernel(feat, edge, W):
    src = edge[0]
    dst = edge[1]
    npad = E_PAD - N_EDGES
    src_pad = jnp.concatenate([src, jnp.zeros((npad,), jnp.int32)])
    dst_pad = jnp.concatenate([dst, jnp.full((npad,), N_NODES, jnp.int32)])
    featflat = jnp.concatenate([feat[:, :DH], feat[:, DH:]], axis=0)
    agg2, deg = _sc_agg(featflat, src_pad, dst_pad)
    deg_bc = jnp.bincount(dst, length=N_PAD).astype(jnp.float32)
    deg_t = jnp.concatenate(
        [deg_bc[:, None], jnp.zeros((N_PAD, NS - 1), jnp.float32)], axis=1)
    return _dense_stage(agg2, deg_t, feat, W)
